# baseline (speedup 1.0000x reference)
import jax
import jax.numpy as jnp
from jax.experimental import pallas as pl
from jax.experimental.pallas import tpu as pltpu

_BS = 64


def _body(colT_ref, rowT_ref, o_hbm, pos, pos1, sem0, sem1):
    d, w = colT_ref.shape
    h = rowT_ref.shape[1]
    colT = colT_ref[...]
    for y in range(h):
        pos[0:d, y * w:(y + 1) * w] = colT
        pos[d:2 * d, y * w:(y + 1) * w] = jnp.broadcast_to(
            rowT_ref[:, y:y + 1], (d, w))
        pos1[0:d, y * w:(y + 1) * w] = colT
        pos1[d:2 * d, y * w:(y + 1) * w] = jnp.broadcast_to(
            rowT_ref[:, y:y + 1], (d, w))
    sems = [sem0, sem1]
    bufs = [pos, pos1]
    copies = [
        pltpu.make_async_copy(bufs[b % 2], o_hbm.at[b], sems[b % 2])
        for b in range(_BS)
    ]
    for b, c in enumerate(copies):
        c.start(priority=b % 2)
    for c in copies:
        c.wait()


def kernel(row_embed, col_embed, bs):
    h, d = row_embed.shape
    w = col_embed.shape[0]
    colT = col_embed.T
    rowT = row_embed.T
    out = pl.pallas_call(
        _body,
        in_specs=[
            pl.BlockSpec((d, w), lambda: (0, 0)),
            pl.BlockSpec((d, h), lambda: (0, 0)),
        ],
        out_specs=pl.BlockSpec(memory_space=pl.ANY),
        out_shape=jax.ShapeDtypeStruct((_BS, 2 * d, h * w), jnp.float32),
        scratch_shapes=[
            pltpu.VMEM((2 * d, h * w), jnp.float32),
            pltpu.VMEM((2 * d, h * w), jnp.float32),
            pltpu.SemaphoreType.DMA,
            pltpu.SemaphoreType.DMA,
        ],
    )(colT, rowT)
    return out.reshape(_BS, 2 * d, h, w)


# speedup vs baseline: 3.8668x; 3.8668x over previous
import jax
import jax.numpy as jnp
from jax.experimental import pallas as pl
from jax.experimental.pallas import tpu as pltpu

_BS = 64


def _body(col_ref, row_ref, o_hbm, pos, sem):
    w, d = col_ref.shape
    h = row_ref.shape[0]
    col = col_ref[...]
    for y in range(h):
        pos[y * w:(y + 1) * w, 0:d] = col
        pos[y * w:(y + 1) * w, d:2 * d] = jnp.broadcast_to(
            row_ref[y:y + 1, :], (w, d))
    copies = [pltpu.make_async_copy(pos, o_hbm.at[b], sem) for b in range(_BS)]
    for c in copies:
        c.start()
    for c in copies:
        c.wait()


def kernel(row_embed, col_embed, bs):
    h, d = row_embed.shape
    w = col_embed.shape[0]
    out = pl.pallas_call(
        _body,
        in_specs=[
            pl.BlockSpec((w, d), lambda: (0, 0)),
            pl.BlockSpec((h, d), lambda: (0, 0)),
        ],
        out_specs=pl.BlockSpec(memory_space=pl.ANY),
        out_shape=jax.ShapeDtypeStruct((_BS, h * w, 2 * d), jnp.float32),
        scratch_shapes=[
            pltpu.VMEM((h * w, 2 * d), jnp.float32),
            pltpu.SemaphoreType.DMA,
        ],
    )(col_embed, row_embed)
    return out.reshape(_BS, h, w, 2 * d).transpose(0, 3, 1, 2)


# speedup vs baseline: 3.8881x; 1.0055x over previous
import jax
import jax.numpy as jnp
from jax.experimental import pallas as pl
from jax.experimental.pallas import tpu as pltpu

_BS = 64


def _body(col_ref, row_ref, o_hbm, pos, sem):
    w, d = col_ref.shape
    h = row_ref.shape[0]
    col = col_ref[...]
    for y in range(h):
        pos[y * w:(y + 1) * w, 0:d] = col
        pos[y * w:(y + 1) * w, d:2 * d] = jnp.broadcast_to(
            row_ref[y:y + 1, :], (w, d))
    copies = [pltpu.make_async_copy(pos, o_hbm.at[b], sem) for b in range(_BS)]
    for b, c in enumerate(copies):
        c.start(priority=b % 2)
    for c in copies:
        c.wait()


def kernel(row_embed, col_embed, bs):
    h, d = row_embed.shape
    w = col_embed.shape[0]
    out = pl.pallas_call(
        _body,
        in_specs=[
            pl.BlockSpec((w, d), lambda: (0, 0)),
            pl.BlockSpec((h, d), lambda: (0, 0)),
        ],
        out_specs=pl.BlockSpec(memory_space=pl.ANY),
        out_shape=jax.ShapeDtypeStruct((_BS, h * w, 2 * d), jnp.float32),
        scratch_shapes=[
            pltpu.VMEM((h * w, 2 * d), jnp.float32),
            pltpu.SemaphoreType.DMA,
        ],
    )(col_embed, row_embed)
    return out.reshape(_BS, h, w, 2 * d).transpose(0, 3, 1, 2)
